# baseline (speedup 1.0000x reference)
import functools

import jax
import jax.numpy as jnp
from jax import lax
from jax.experimental import pallas as pl
from jax.experimental.pallas import tpu as pltpu
from jax.experimental.pallas import tpu_sc as plsc

HIDDEN = 512
EPS = 1e-12
_NSLOT = 4
_SPAD = 1280
_NW = 32
_BPW = _SPAD // _NW


def _sc_gather(table_hbm, idx_hbm, out_hbm, idx_v, rows_v, sem):
    wid = lax.axis_index("s") * 2 + lax.axis_index("c")
    base = wid * _BPW
    pltpu.sync_copy(idx_hbm.at[pl.ds(base, _BPW)], idx_v)
    pltpu.async_copy(table_hbm.at[idx_v], rows_v, sem).wait()
    pltpu.sync_copy(rows_v, out_hbm.at[pl.ds(base, _BPW)])


def _tc_body(x_hbm, pos_hbm, g_ref, b_ref, out_hbm,
             x_buf, o_buf, p_buf, in_sems, out_sems, pos_sem):
    B = x_hbm.shape[0]
    S = x_hbm.shape[1]

    def in_copy(b, slot):
        return pltpu.make_async_copy(x_hbm.at[b], x_buf.at[slot],
                                     in_sems.at[slot])

    def out_copy(b, slot):
        return pltpu.make_async_copy(o_buf.at[slot], out_hbm.at[b],
                                     out_sems.at[slot])

    pos_cp = pltpu.make_async_copy(pos_hbm, p_buf, pos_sem)
    pos_cp.start()
    for b0 in range(_NSLOT):
        in_copy(b0, b0).start()
    pos_cp.wait()

    g = g_ref[...]
    bt = b_ref[...]
    p = p_buf[...][:S]

    def b_step(b, carry):
        slot = lax.rem(b, _NSLOT)
        in_copy(b, slot).wait()

        @pl.when(b >= _NSLOT)
        def _():
            out_copy(b, slot).wait()

        e = x_buf[slot] + p
        m = jnp.mean(e, axis=-1, keepdims=True)
        d = e - m
        v = jnp.mean(d * d, axis=-1, keepdims=True)
        o_buf[slot] = d * lax.rsqrt(v + EPS) * g + bt

        out_copy(b, slot).start()

        @pl.when(b + _NSLOT < B)
        def _():
            in_copy(b + _NSLOT, slot).start()
        return carry

    lax.fori_loop(0, B, b_step, 0)

    for b in range(B - _NSLOT, B):
        out_copy(b, b % _NSLOT).wait()


def kernel(inputs_embeds, position_embeddings, gamma, beta, position_ids,
           past_key_values_length):
    B, S, H = inputs_embeds.shape
    table = position_embeddings[0]

    pos_ids = lax.dynamic_slice_in_dim(
        position_ids, past_key_values_length, S, axis=0).astype(jnp.int32)
    ids_pad = jnp.concatenate(
        [pos_ids, jnp.zeros((_SPAD - S,), jnp.int32)])

    mesh = plsc.VectorSubcoreMesh(core_axis_name="c", subcore_axis_name="s")
    pos_g = functools.partial(
        pl.kernel, mesh=mesh,
        out_type=jax.ShapeDtypeStruct((_SPAD, H), jnp.float32),
        scratch_types=[
            pltpu.VMEM((_BPW,), jnp.int32),
            pltpu.VMEM((_BPW, H), jnp.float32),
            pltpu.SemaphoreType.DMA,
        ],
    )(_sc_gather)(table, ids_pad)

    g2 = gamma.reshape(1, H)
    b2 = beta.reshape(1, H)

    out = pl.pallas_call(
        _tc_body,
        in_specs=[
            pl.BlockSpec(memory_space=pl.ANY),
            pl.BlockSpec(memory_space=pl.ANY),
            pl.BlockSpec(memory_space=pltpu.VMEM),
            pl.BlockSpec(memory_space=pltpu.VMEM),
        ],
        out_specs=pl.BlockSpec(memory_space=pl.ANY),
        out_shape=jax.ShapeDtypeStruct((B, S, H), jnp.float32),
        scratch_shapes=[
            pltpu.VMEM((_NSLOT, S, H), jnp.float32),
            pltpu.VMEM((_NSLOT, S, H), jnp.float32),
            pltpu.VMEM((_SPAD, H), jnp.float32),
            pltpu.SemaphoreType.DMA((_NSLOT,)),
            pltpu.SemaphoreType.DMA((_NSLOT,)),
            pltpu.SemaphoreType.DMA,
        ],
    )(inputs_embeds, pos_g, g2, b2)
    return out
